# Initial kernel scaffold; baseline (speedup 1.0000x reference)
import jax
import jax.numpy as jnp
from jax.experimental import pallas as pl


def kernel(x, table):
    raise NotImplementedError("write your pallas kernel here")


# speedup vs baseline: 1.6155x; 1.6155x over previous
import functools

import jax
import jax.numpy as jnp
import numpy as np
from jax import lax
from jax.experimental import pallas as pl
from jax.experimental.pallas import tpu as pltpu
from jax.experimental.pallas import tpu_sc as plsc

_D = 128
_SCALE = float(np.sqrt(_D))
_NC, _NS, _L = 2, 16, 16
_NW = _NC * _NS


def _pos_table(seq_len: int) -> jax.Array:
    pos = jnp.arange(seq_len, dtype=jnp.float32)[:, None]
    i2 = jnp.arange(0, _D, 2, dtype=jnp.float32)
    ang = pos / jnp.power(10000.0, i2 / float(_D))
    enc = jnp.zeros((seq_len, _D), dtype=jnp.float32)
    enc = enc.at[:, 0::2].set(jnp.sin(ang))
    enc = enc.at[:, 1::2].set(jnp.cos(ang))
    return enc


def kernel(x, table):
    B, S = x.shape
    N = B * S
    b_per_w = N // _NW
    C = 128
    nchunk = b_per_w // C
    assert N % _NW == 0 and b_per_w % C == 0 and S % b_per_w == 0

    pos = _pos_table(S)
    xw = x.reshape(_NW, nchunk, C)

    mesh = plsc.VectorSubcoreMesh(
        core_axis_name="c", subcore_axis_name="s",
        num_cores=_NC, num_subcores=_NS,
    )

    @functools.partial(
        pl.kernel,
        out_type=jax.ShapeDtypeStruct((N, _D), jnp.float32),
        mesh=mesh,
        scratch_types=[
            pltpu.VMEM((nchunk, C), jnp.int32),
            pltpu.VMEM((2, C, _D), jnp.float32),
            pltpu.VMEM((b_per_w, _D), jnp.float32),
            pltpu.SemaphoreType.DMA,
            pltpu.SemaphoreType.DMA,
        ],
    )
    def emb_kernel(x_hbm, table_hbm, pos_hbm, out_hbm,
                   idx_v, rows_v, pos_v, sem0, sem1):
        wid = lax.axis_index("s") * _NC + lax.axis_index("c")
        base = wid * b_per_w
        pos_base = lax.rem(base, S)

        pltpu.sync_copy(x_hbm.at[wid], idx_v)
        pltpu.sync_copy(pos_hbm.at[pl.ds(pos_base, b_per_w)], pos_v)

        sems = (sem0, sem1)
        descs = [pltpu.async_copy(table_hbm.at[idx_v.at[0]], rows_v.at[0],
                                  sems[0])]
        for c in range(nchunk):
            slot = c % 2
            descs[c].wait()
            if c + 1 < nchunk:
                descs.append(
                    pltpu.async_copy(table_hbm.at[idx_v.at[c + 1]],
                                     rows_v.at[(c + 1) % 2],
                                     sems[(c + 1) % 2]))

            def body(i, _, slot=slot, c=c):
                for j in range(_D // _L):
                    g = rows_v[slot, i, pl.ds(j * _L, _L)]
                    p = pos_v[c * C + i, pl.ds(j * _L, _L)]
                    rows_v[slot, i, pl.ds(j * _L, _L)] = g * _SCALE + p
                return 0

            lax.fori_loop(0, C, body, 0)
            pltpu.sync_copy(rows_v.at[slot],
                            out_hbm.at[pl.ds(base + c * C, C)])

    out = emb_kernel(xw, table, pos)
    return out.reshape(B, S, _D)


# speedup vs baseline: 1.6614x; 1.0284x over previous
import functools

import jax
import jax.numpy as jnp
import numpy as np
from jax import lax
from jax.experimental import pallas as pl
from jax.experimental.pallas import tpu as pltpu
from jax.experimental.pallas import tpu_sc as plsc

_D = 128
_SCALE = float(np.sqrt(_D))
_NC, _NS, _L = 2, 16, 16
_NW = _NC * _NS


def _pos_table(seq_len: int) -> jax.Array:
    pos = jnp.arange(seq_len, dtype=jnp.float32)[:, None]
    i2 = jnp.arange(0, _D, 2, dtype=jnp.float32)
    ang = pos / jnp.power(10000.0, i2 / float(_D))
    enc = jnp.zeros((seq_len, _D), dtype=jnp.float32)
    enc = enc.at[:, 0::2].set(jnp.sin(ang))
    enc = enc.at[:, 1::2].set(jnp.cos(ang))
    return enc


def kernel(x, table):
    B, S = x.shape
    N = B * S
    C = S // _NW
    assert S % _NW == 0 and _D % _L == 0

    pos_div = _pos_table(S) * np.float32(1.0 / _SCALE)
    xw = x.reshape(B, _NW, C).transpose(1, 0, 2)

    mesh = plsc.VectorSubcoreMesh(
        core_axis_name="c", subcore_axis_name="s",
        num_cores=_NC, num_subcores=_NS,
    )

    @functools.partial(
        pl.kernel,
        out_type=jax.ShapeDtypeStruct((N, _D), jnp.float32),
        mesh=mesh,
        scratch_types=[
            pltpu.VMEM((B, C), jnp.int32),
            pltpu.VMEM((B, C, _D), jnp.float32),
            [pltpu.SemaphoreType.DMA] * 4,
            [pltpu.SemaphoreType.DMA] * 4,
            [pltpu.SemaphoreType.DMA] * 4,
        ],
    )
    def emb_kernel(x_hbm, table_hbm, pos_hbm, out_hbm,
                   idx_v, rows_v, psems, gsems, wsems):
        wid = lax.axis_index("s") * _NC + lax.axis_index("c")
        ws = wid * C

        pltpu.sync_copy(x_hbm.at[wid], idx_v)
        pos_descs = [
            pltpu.async_copy(pos_hbm.at[pl.ds(ws, C)], rows_v.at[c], psems[c])
            for c in range(B)
        ]
        gadd_descs = []
        for c in range(B):
            pos_descs[c].wait()
            gadd_descs.append(
                pltpu.async_copy(table_hbm.at[idx_v.at[c]], rows_v.at[c],
                                 gsems[c], add=True))
        wb_descs = []
        for c in range(B):
            gadd_descs[c].wait()

            def body(i, _, c=c):
                for j in range(_D // _L):
                    sl = pl.ds(j * _L, _L)
                    rows_v[c, i, sl] = rows_v[c, i, sl] * _SCALE
                return 0

            lax.fori_loop(0, C, body, 0)
            wb_descs.append(
                pltpu.async_copy(rows_v.at[c],
                                 out_hbm.at[pl.ds(c * S + ws, C)], wsems[c]))
        for c in range(B):
            wb_descs[c].wait()

    out = emb_kernel(xw, table, pos_div)
    return out.reshape(B, S, _D)
